# baseline (speedup 1.0000x reference)
import functools

import jax
import jax.numpy as jnp
from jax import lax
from jax.experimental import pallas as pl
from jax.experimental.pallas import tpu as pltpu
from jax.experimental.pallas import tpu_sc as plsc

D = 64
TABLE_ROWS = 200
SEQ = 200
NC = 2
NS = 16
NW = NC * NS
G1 = 128
G2 = SEQ - G1
IDX_ROWS = 8
NB = 8
PF = 4
PAIR = 2 * IDX_ROWS


def _combine_body(v_ref, p_ref, o_ref):
    o_ref[...] = jnp.broadcast_to(
        (v_ref[...] + p_ref[...])[None], (NW, TABLE_ROWS, D)
    )


def _combine_tables(vocab_slice, pos_table):
    return pl.pallas_call(
        _combine_body,
        out_shape=jax.ShapeDtypeStruct((NW, TABLE_ROWS, D), jnp.float32),
    )(vocab_slice, pos_table)


def _gather_body(b_pw, comb, x_hbm, out, idx_v, rows_v, gsem, ssem, isem):
    wid = lax.axis_index("s") * NC + lax.axis_index("c")
    base = wid * b_pw
    npairs = b_pw // PAIR
    myt = comb.at[wid]

    def fire_gather(s, ib, j2):
        pltpu.async_copy(myt.at[idx_v.at[ib, j2, pl.ds(0, G1)]],
                         rows_v.at[s, pl.ds(0, G1)], gsem.at[s])
        pltpu.async_copy(myt.at[idx_v.at[ib, j2, pl.ds(G1, G2)]],
                         rows_v.at[s, pl.ds(G1, G2)], gsem.at[s])

    def wait_gather(s, ib, j2):
        pltpu.make_async_copy(myt.at[idx_v.at[ib, j2, pl.ds(0, G1)]],
                              rows_v.at[s, pl.ds(0, G1)], gsem.at[s]).wait()
        pltpu.make_async_copy(myt.at[idx_v.at[ib, j2, pl.ds(G1, G2)]],
                              rows_v.at[s, pl.ds(G1, G2)], gsem.at[s]).wait()

    def fire_store(b, s):
        pltpu.async_copy(rows_v.at[s], out.at[b], ssem.at[s])

    def wait_store(b, s):
        pltpu.make_async_copy(rows_v.at[s], out.at[b], ssem.at[s]).wait()

    def fire_idx(first_b, ib):
        pltpu.async_copy(x_hbm.at[pl.ds(first_b, IDX_ROWS)], idx_v.at[ib],
                         isem.at[ib])

    def wait_idx(first_b, ib):
        pltpu.make_async_copy(x_hbm.at[pl.ds(first_b, IDX_ROWS)],
                              idx_v.at[ib], isem.at[ib]).wait()

    def step(pb, p_has_next, j, first_pair, last_pair):
        s = j % NB
        ib, j2 = divmod(j, IDX_ROWS)
        b = pb + j
        wait_gather(s, ib, j2)
        fire_store(b, s)
        jp = (j + PF) % PAIR
        ibp, j2p = divmod(jp, IDX_ROWS)
        sp = (j + PF) % NB
        if not (last_pair and j >= PAIR - PF):
            if j == IDX_ROWS - PF:
                wait_idx(pb + IDX_ROWS, 1)
            if j == PAIR - PF:
                wait_idx(pb + PAIR, 0)
            if not (first_pair and j < PF):
                wait_store(b + PF - NB, sp)
            fire_gather(sp, ibp, j2p)
        else:
            wait_store(b + PF - NB, sp)
        if j == IDX_ROWS - 1 and p_has_next:
            fire_idx(pb + PAIR, 0)
        if j == PAIR - 1 and p_has_next:
            fire_idx(pb + PAIR + IDX_ROWS, 1)

    pltpu.sync_copy(x_hbm.at[pl.ds(base, IDX_ROWS)], idx_v.at[0])
    fire_idx(base + IDX_ROWS, 1)
    for j in range(PF):
        fire_gather(j, 0, j)

    for j in range(PAIR):
        step(base, True, j, True, False)

    def pair_body(p, _):
        pb = base + p * PAIR
        for j in range(PAIR):
            step(pb, True, j, False, False)
        return 0

    lax.fori_loop(1, npairs - 1, pair_body, 0, unroll=False)

    lb = base + (npairs - 1) * PAIR
    for j in range(PAIR):
        step(lb, False, j, False, True)

    for j in range(PAIR - PF, PAIR):
        wait_store(lb + j, j % NB)


def kernel(x, vocab_table, pos_table):
    B, S = x.shape
    assert S == SEQ and B % (NW * PAIR) == 0
    b_pw = B // NW

    combined = _combine_tables(
        lax.slice(vocab_table, (0, 0), (TABLE_ROWS, D)), pos_table
    )

    mesh = plsc.VectorSubcoreMesh(core_axis_name="c", subcore_axis_name="s")
    out = pl.kernel(
        functools.partial(_gather_body, b_pw),
        out_type=jax.ShapeDtypeStruct((B, S, D), jnp.float32),
        mesh=mesh,
        scratch_types=[
            pltpu.VMEM((2, IDX_ROWS, SEQ), jnp.int32),
            pltpu.VMEM((NB, SEQ, D), jnp.float32),
            pltpu.SemaphoreType.DMA((NB,)),
            pltpu.SemaphoreType.DMA((NB,)),
            pltpu.SemaphoreType.DMA((2,)),
        ],
        compiler_params=pltpu.CompilerParams(use_tc_tiling_on_sc=False),
    )(combined, x.astype(jnp.int32))

    return out


# speedup vs baseline: 1.2923x; 1.2923x over previous
import functools

import jax
import jax.numpy as jnp
from jax import lax
from jax.experimental import pallas as pl
from jax.experimental.pallas import tpu as pltpu
from jax.experimental.pallas import tpu_sc as plsc

D = 64
TABLE_ROWS = 200
SEQ = 200
NC = 2
NS = 16
NW = NC * NS
NTR = D // 8
SPW = SEQ // (NW // NTR)
TCW = 16
CB = TCW * 128
L = 16


def _combine_body(v_ref, p_ref, o_ref):
    o_ref[...] = (v_ref[...] + p_ref[...]).T


def _combine_tables(vocab_slice, pos_table):
    return pl.pallas_call(
        _combine_body,
        out_shape=jax.ShapeDtypeStruct((D, TABLE_ROWS), jnp.float32),
    )(vocab_slice, pos_table)


def _gather_body(nb, tt_hbm, xt_hbm, out, tt_v, idx_v, buf_v, isem, ssem):
    wid = lax.axis_index("s") * NC + lax.axis_index("c")
    tr = wid % NTR
    s0 = (wid // NTR) * SPW
    nchunks = SPW * (nb // TCW)
    cpp = nb // TCW

    pltpu.sync_copy(tt_hbm, tt_v)
    doff = [
        jnp.full((L,), 0, jnp.int32) + (tr * 8 + dm) * TABLE_ROWS
        for dm in range(8)
    ]

    def idx_src(c):
        s = s0 + c // cpp
        b0 = (c % cpp) * CB
        return xt_hbm.at[s, pl.ds(b0, CB)]

    def out_dst(c):
        s = s0 + c // cpp
        tc0 = (c % cpp) * TCW
        return out.at[s, tr, pl.ds(tc0, TCW)]

    def fire_idx(c, e):
        pltpu.async_copy(idx_src(c), idx_v.at[e], isem.at[e])

    def wait_idx(c, e):
        pltpu.make_async_copy(idx_src(c), idx_v.at[e], isem.at[e]).wait()

    def fire_store(c, e):
        pltpu.async_copy(buf_v.at[e], out_dst(c), ssem.at[e])

    def wait_store(c, e):
        pltpu.make_async_copy(buf_v.at[e], out_dst(c), ssem.at[e]).wait()

    def compute(e):
        def tile_body(j, _):
            for g in range(8):
                idxv = idx_v[e, pl.ds(j * 128 + g * L, L)]
                for dm in range(8):
                    val = plsc.load_gather(tt_v, [idxv + doff[dm]])
                    buf_v[e, j, dm, pl.ds(g * L, L)] = val
            return 0

        lax.fori_loop(0, TCW, tile_body, 0, unroll=False)

    fire_idx(0, 0)
    fire_idx(1, 1)

    wait_idx(0, 0)
    compute(0)
    fire_store(0, 0)
    fire_idx(2, 0)
    wait_idx(1, 1)
    compute(1)
    fire_store(1, 1)
    fire_idx(3, 1)

    def pair_body(p, _):
        c = 2 * p
        for e in range(2):
            wait_idx(c + e, e)
            wait_store(c + e - 2, e)
            compute(e)
            fire_store(c + e, e)
            fire_idx(c + e + 2, e)
        return 0

    lax.fori_loop(1, nchunks // 2 - 1, pair_body, 0, unroll=False)

    cl = nchunks - 2
    for e in range(2):
        wait_idx(cl + e, e)
        wait_store(cl + e - 2, e)
        compute(e)
        fire_store(cl + e, e)

    for e in range(2):
        wait_store(cl + e, e)


def kernel(x, vocab_table, pos_table):
    B, S = x.shape
    assert S == SEQ and B % 128 == 0
    nb = B // 128

    tt = _combine_tables(
        lax.slice(vocab_table, (0, 0), (TABLE_ROWS, D)), pos_table
    )
    tt_flat = tt.reshape(D * TABLE_ROWS)

    xt = x.T.astype(jnp.int32)

    mesh = plsc.VectorSubcoreMesh(core_axis_name="c", subcore_axis_name="s")
    buf5 = pl.kernel(
        functools.partial(_gather_body, nb),
        out_type=jax.ShapeDtypeStruct((SEQ, NTR, nb, 8, 128), jnp.float32),
        mesh=mesh,
        scratch_types=[
            pltpu.VMEM((D * TABLE_ROWS,), jnp.float32),
            pltpu.VMEM((2, CB), jnp.int32),
            pltpu.VMEM((2, TCW, 8, 128), jnp.float32),
            pltpu.SemaphoreType.DMA((2,)),
            pltpu.SemaphoreType.DMA((2,)),
        ],
        compiler_params=pltpu.CompilerParams(
            use_tc_tiling_on_sc=False, needs_layout_passes=False
        ),
    )(tt_flat, xt)

    return buf5.transpose(2, 4, 0, 1, 3).reshape(B, SEQ, D)
